# baseline (speedup 1.0000x reference)
import functools

import jax
import jax.numpy as jnp
from jax import lax
from jax.experimental import pallas as pl
from jax.experimental.pallas import tpu as pltpu
from jax.experimental.pallas import tpu_sc as plsc

_N = 100000
_E = 3200000
_B = 10
_NC = 2
_NS = 16
_NW = _NC * _NS
_NP = 100096
_R = _NP // 128
_EPT = _E // _NW
_K = 4000
_NWIN = _EPT // _K
_SL = _NP // _NS

def _mesh():
    return plsc.VectorSubcoreMesh(
        core_axis_name="c", subcore_axis_name="s",
        num_cores=_NC, num_subcores=_NS)


def _lk(y):
    return jnp.where(y > 0, y, 0.01 * y)


def _k_mm_leaky(a_ref, b_ref, o_ref):
    o_ref[...] = _lk(jnp.dot(a_ref[...], b_ref[...],
                             preferred_element_type=jnp.float32))


def _k_mm_bias_leaky(a_ref, b_ref, c_ref, o_ref):
    o_ref[...] = _lk(jnp.dot(a_ref[...], b_ref[...],
                             preferred_element_type=jnp.float32) + c_ref[...])


def _k_maxlead(t_ref, o_ref):
    o_ref[...] = jnp.max(t_ref[...], axis=0)


def _k_head(t_ref, w_ref, b_ref, o_ref):
    m = jnp.max(t_ref[...], axis=0)
    o_ref[...] = jax.nn.sigmoid(
        jnp.dot(m, w_ref[...], preferred_element_type=jnp.float32) + b_ref[...])


def _k_norm1(d0, d1, x0, x1, r0, r1, indeg_o, dis1_o, p0_o, p1_o, p2_o, p3_o):
    indeg = d0[...] + d1[...]
    dis1 = lax.rsqrt(indeg + 1.0)
    indeg_o[...] = indeg
    dis1_o[...] = dis1
    p0_o[...] = dis1 * x0[...]
    p1_o[...] = dis1 * x1[...]
    p2_o[...] = dis1 * r0[...]
    p3_o[...] = dis1 * r1[...]


def _k_mid(sa0, sa1, sa2, sa3, sb0, sb1, sb2, sb3, x0, x1, r0, r1,
           indeg_r, w1_r, b1_r, w2_r, q_o, dis2_o):
    indeg = indeg_r[...]
    dis1 = lax.rsqrt(indeg + 1.0)
    cnt = indeg + 1.0
    h = (x0[...], x1[...], r0[...], r1[...])
    s = (sa0[...] + sb0[...], sa1[...] + sb1[...],
         sa2[...] + sb2[...], sa3[...] + sb3[...])
    agg = [dis1 * s[k] + dis1 * dis1 * h[k] for k in range(4)]
    h1w = jnp.zeros_like(indeg)
    for j in range(8):
        oj = (agg[0] * w1_r[0, j] + agg[1] * w1_r[1, j]
              + agg[2] * w1_r[2, j] + agg[3] * w1_r[3, j]) / cnt + b1_r[0, j]
        h1w = h1w + _lk(oj) * w2_r[j, 0]
    dis2 = lax.rsqrt(indeg + 2.0)
    q_o[...] = dis2 * h1w
    dis2_o[...] = dis2


def _k_fin(mall, qq, dd, b2_r, o_ref):
    z = dd[...] * jnp.maximum(jnp.max(mall[...], axis=0),
                              2.0 * qq[...]) + b2_r[0, 0]
    z = z - jnp.max(z, axis=1, keepdims=True)
    e = jnp.exp(z)
    o_ref[...] = e / jnp.sum(e, axis=1, keepdims=True)


def _tc(fn, out_shapes, *args, smem_args=0):
    n = len(args)
    in_specs = [pl.BlockSpec(memory_space=pltpu.VMEM)] * (n - smem_args) + \
               [pl.BlockSpec(memory_space=pltpu.SMEM)] * smem_args
    return pl.pallas_call(fn, out_shape=out_shapes, in_specs=in_specs)(*args)


def _wid_base():
    cid = lax.axis_index("c")
    sid = lax.axis_index("s")
    wid = sid * _NC + cid
    return cid, sid, wid * _EPT


def _deg_body(col_h, z_h, ones_h, out_h, idx_v, ones_v, obuf, acc, sem):
    cid, sid, base = _wid_base()
    o = sid * _SL
    pltpu.sync_copy(z_h.at[pl.ds(o, _SL)], obuf)
    pltpu.sync_copy(obuf, acc.at[pl.ds(o, _SL)])
    pltpu.sync_copy(ones_h, ones_v)
    plsc.subcore_barrier()

    def wb(w, carry):
        pltpu.sync_copy(col_h.at[pl.ds(base + w * _K, _K)], idx_v)
        pltpu.sync_copy(ones_v, acc.at[idx_v], add=True)
        return carry

    lax.fori_loop(0, _NWIN, wb, 0)
    plsc.subcore_barrier()
    pltpu.sync_copy(acc.at[pl.ds(o, _SL)], obuf)
    pltpu.sync_copy(obuf, out_h.at[pl.ds(cid * _NP + o, _SL)])


def _run_deg(col, zeros_np, ones_k):
    f = functools.partial(
        pl.kernel, _deg_body,
        out_type=jax.ShapeDtypeStruct((_NC * _NP,), jnp.float32),
        mesh=_mesh(),
        scratch_types=[
            pltpu.VMEM((_K,), jnp.int32),
            pltpu.VMEM((_K,), jnp.float32),
            pltpu.VMEM((_SL,), jnp.float32),
            pltpu.VMEM_SHARED((_NP,), jnp.float32),
            pltpu.SemaphoreType.DMA,
        ],
    )()
    return f(col, zeros_np, ones_k)


def _sum_body(row_h, col_h, h0_h, h1_h, h2_h, h3_h, z_h, ones_h,
              o0_h, o1_h, o2_h, o3_h, oi_h,
              idxr, idxc, vv0, vv1, vv2, vv3, ones_v, obuf, ybuf, hbuf,
              accd, sp0, sp1, sp2, sp3,
              ac0, ac1, ac2, ac3, sem):
    cid, sid, base = _wid_base()
    o = sid * _SL
    sl = pl.ds(o, _SL)
    pltpu.sync_copy(z_h.at[sl], obuf)
    pltpu.sync_copy(obuf, accd.at[sl])
    for ac in (ac0, ac1, ac2, ac3):
        pltpu.sync_copy(obuf, ac.at[sl])
    pltpu.sync_copy(ones_h, ones_v)
    plsc.subcore_barrier()

    base_a = sid * (_E // _NS)

    def wa(w, carry):
        pltpu.sync_copy(col_h.at[pl.ds(base_a + w * _K, _K)], idxc)
        pltpu.sync_copy(ones_v, accd.at[idxc], add=True)
        return carry

    lax.fori_loop(0, (_E // _NS) // _K, wa, 0)
    plsc.subcore_barrier()

    pltpu.sync_copy(accd.at[sl], obuf)
    pltpu.sync_copy(obuf, oi_h.at[pl.ds(cid * _NP + o, _SL)])

    def nb(j, carry):
        s16 = pl.ds(j * 16, 16)
        d = obuf[s16] + 1.0
        i = plsc.bitcast(d, jnp.int32)
        i = 0x5F3759DF - lax.shift_right_logical(i, 1)
        y = plsc.bitcast(i, jnp.float32)
        for _ in range(3):
            y = y * (1.5 - 0.5 * d * y * y)
        ybuf[s16] = y
        return carry

    lax.fori_loop(0, _SL // 16, nb, 0)
    for h_h, sp in ((h0_h, sp0), (h1_h, sp1), (h2_h, sp2), (h3_h, sp3)):
        pltpu.sync_copy(h_h.at[sl], hbuf)

        def mv(j, carry):
            s16 = pl.ds(j * 16, 16)
            hbuf[s16] = hbuf[s16] * ybuf[s16]
            return carry

        lax.fori_loop(0, _SL // 16, mv, 0)
        pltpu.sync_copy(hbuf, sp.at[sl])
    plsc.subcore_barrier()

    def wb(w, carry):
        off = base + w * _K
        pltpu.sync_copy(row_h.at[pl.ds(off, _K)], idxr)
        pltpu.sync_copy(col_h.at[pl.ds(off, _K)], idxc)
        vvs = (vv0, vv1, vv2, vv3)
        g = [pltpu.async_copy(sp.at[idxr], v, sem)
             for sp, v in zip((sp0, sp1, sp2, sp3), vvs)]
        for d in g:
            d.wait()
        s = [pltpu.async_copy(v, ac.at[idxc], sem, add=True)
             for ac, v in zip((ac0, ac1, ac2, ac3), vvs)]
        for d in s:
            d.wait()
        return carry

    lax.fori_loop(0, _NWIN, wb, 0)
    plsc.subcore_barrier()
    out_sl = pl.ds(cid * _NP + o, _SL)
    for ac, o_h in ((ac0, o0_h), (ac1, o1_h), (ac2, o2_h), (ac3, o3_h)):
        pltpu.sync_copy(ac.at[sl], obuf)
        pltpu.sync_copy(obuf, o_h.at[out_sl])


def _run_sum(row, col, h0, h1, h2, h3, zeros_np, ones_k):
    st = jax.ShapeDtypeStruct((_NC * _NP,), jnp.float32)
    f = functools.partial(
        pl.kernel, _sum_body,
        out_type=(st, st, st, st, st),
        mesh=_mesh(),
        compiler_params=pltpu.CompilerParams(needs_layout_passes=False),
        scratch_types=[
            pltpu.VMEM((_K,), jnp.int32),
            pltpu.VMEM((_K,), jnp.int32),
            pltpu.VMEM((_K,), jnp.float32),
            pltpu.VMEM((_K,), jnp.float32),
            pltpu.VMEM((_K,), jnp.float32),
            pltpu.VMEM((_K,), jnp.float32),
            pltpu.VMEM((_K,), jnp.float32),
            pltpu.VMEM((_SL,), jnp.float32),
            pltpu.VMEM((_SL,), jnp.float32),
            pltpu.VMEM((_SL,), jnp.float32),
            pltpu.VMEM_SHARED((_NP,), jnp.float32),
            pltpu.VMEM_SHARED((_NP,), jnp.float32),
            pltpu.VMEM_SHARED((_NP,), jnp.float32),
            pltpu.VMEM_SHARED((_NP,), jnp.float32),
            pltpu.VMEM_SHARED((_NP,), jnp.float32),
            pltpu.VMEM_SHARED((_NP,), jnp.float32),
            pltpu.VMEM_SHARED((_NP,), jnp.float32),
            pltpu.VMEM_SHARED((_NP,), jnp.float32),
            pltpu.VMEM_SHARED((_NP,), jnp.float32),
            pltpu.SemaphoreType.DMA,
        ],
    )()
    return f(row, col, h0, h1, h2, h3, zeros_np, ones_k)


_SL2 = _SL // 2
_TMP = 8192


def _max_body(row_h, col_h, q_h, out_h, idxr, idxc, qv, kbuf, vbuf, tmp,
              hist, sq, abuf, sem):
    cid, sid, base = _wid_base()
    neg = jnp.float32(-3.4e38)
    o = sid * _SL
    for half in range(2):
        o2 = pl.ds(o + half * _SL2, _SL2)
        pltpu.sync_copy(q_h.at[o2], abuf)
        pltpu.sync_copy(abuf, sq.at[o2])
    plsc.subcore_barrier()

    def ib(j, carry):
        hist[pl.ds(j * 16, 16)] = jnp.full((16,), neg, jnp.float32)
        return carry

    lax.fori_loop(0, _NP // 16, ib, 0)

    def wb(w, carry):
        off = base + w * _K
        pltpu.sync_copy(row_h.at[pl.ds(off, _K)], idxr)
        pltpu.sync_copy(col_h.at[pl.ds(off, _K)], idxc)
        pltpu.async_copy(sq.at[idxr], qv, sem).wait()

        lane = lax.iota(jnp.int32, 16)

        def vb(j, c2):
            ck = idxc[pl.ds(j * 16, 16)]
            vv = qv[pl.ds(j * 16, 16)]
            hk = lax.bitwise_and(ck, _TMP - 1)
            lane_f = lane.astype(jnp.float32)
            plsc.store_scatter(tmp, [hk], lane_f)
            w = plsc.load_gather(tmp, [hk])
            dup = jnp.any(w != lane_f)

            @pl.when(jnp.logical_not(dup))
            def _():
                cur = plsc.load_gather(hist, [ck])
                plsc.store_scatter(hist, [ck], jnp.maximum(cur, vv))

            @pl.when(dup)
            def _():
                ck_f = ck.astype(jnp.float32)
                kbuf[...] = ck_f
                vbuf[...] = vv
                acc = vv
                for r in range(1, 16):
                    ir = lax.bitwise_and(lane + r, 15)
                    kr = plsc.load_gather(kbuf, [ir])
                    vr = plsc.load_gather(vbuf, [ir])
                    acc = jnp.where(kr == ck_f, jnp.maximum(acc, vr), acc)
                cur = plsc.load_gather(hist, [ck])
                plsc.store_scatter(hist, [ck], jnp.maximum(cur, acc))

            return c2

        lax.fori_loop(0, _K // 16, vb, 0)
        return carry

    lax.fori_loop(0, _NWIN, wb, 0)
    wid = sid * _NC + cid
    pltpu.sync_copy(hist, out_h.at[pl.ds(wid * _NP, _NP)])


def _run_max(row, col, qtab):
    f = functools.partial(
        pl.kernel, _max_body,
        out_type=jax.ShapeDtypeStruct((_NW * _NP,), jnp.float32),
        mesh=_mesh(),
        compiler_params=pltpu.CompilerParams(needs_layout_passes=False),
        scratch_types=[
            pltpu.VMEM((_K,), jnp.int32),
            pltpu.VMEM((_K,), jnp.int32),
            pltpu.VMEM((_K,), jnp.float32),
            pltpu.VMEM((16,), jnp.float32),
            pltpu.VMEM((16,), jnp.float32),
            pltpu.VMEM((_TMP,), jnp.float32),
            pltpu.VMEM((_NP,), jnp.float32),
            pltpu.VMEM_SHARED((_NP,), jnp.float32),
            pltpu.VMEM((_SL2,), jnp.float32),
            pltpu.SemaphoreType.DMA,
        ],
    )()
    return f(row, col, qtab)


def kernel(im, x, edge_index, num_nodes_per_graph, conv1_w, conv2_w, conv2_b,
           lin_w, lin_b, g1_W, g1_b, g2_W, g2_b):
    f32 = jnp.float32

    im_p = jnp.pad(im, ((0, 0), (0, 0), (2, 2), (2, 2)))
    t1 = [im_p[:, :, ky:ky + 57:4, kx:kx + 57:4]
          for ky in range(11) for kx in range(11)]
    pat1 = jnp.stack(t1, 0).transpose(1, 3, 4, 2, 0).reshape(_B * 225, 363)
    c1 = _tc(_k_mm_leaky, jax.ShapeDtypeStruct((_B * 225, 10), f32),
             pat1, conv1_w.reshape(10, 363).T)
    c1r = c1.reshape(_B, 15, 15, 10)
    t2 = [c1r[:, dy:dy + 13:2, dx:dx + 13:2, :]
          for dy in range(3) for dx in range(3)]
    p1 = _tc(_k_maxlead, jax.ShapeDtypeStruct((_B * 49, 10), f32),
             jnp.stack(t2, 0).reshape(9, _B * 49, 10))
    p1p = jnp.pad(p1.reshape(_B, 7, 7, 10).transpose(0, 3, 1, 2),
                  ((0, 0), (0, 0), (2, 2), (2, 2)))
    t3 = [p1p[:, :, ky:ky + 7:2, kx:kx + 7:2]
          for ky in range(5) for kx in range(5)]
    pat2 = jnp.stack(t3, 0).transpose(1, 3, 4, 2, 0).reshape(_B * 16, 250)
    c2 = _tc(_k_mm_bias_leaky, jax.ShapeDtypeStruct((_B * 16, 16), f32),
             pat2, conv2_w.reshape(16, 250).T, conv2_b.reshape(1, 16))
    c2r = c2.reshape(_B, 4, 4, 16)
    t4 = [c2r[:, dy:dy + 3:2, dx:dx + 3:2, :].transpose(0, 3, 1, 2).reshape(_B, 64)
          for dy in range(2) for dx in range(2)]
    imf = _tc(_k_head, jax.ShapeDtypeStruct((_B, 2), f32),
              jnp.stack(t4, 0), lin_w.T, lin_b.reshape(1, 2))

    row = edge_index[0]
    col = edge_index[1]
    rep = jnp.repeat(imf, num_nodes_per_graph, axis=0, total_repeat_length=_N)
    pad = _NP - _N

    def nshape(v):
        return jnp.pad(v, (0, pad)).reshape(_R, 128)

    x0, x1 = nshape(x[:, 0]), nshape(x[:, 1])
    r0, r1 = nshape(rep[:, 0]), nshape(rep[:, 1])

    zeros_np = jnp.zeros((_NP,), f32)
    ones_k = jnp.ones((_K,), f32)

    sh = jax.ShapeDtypeStruct((_R, 128), f32)
    sp = _run_sum(row, col,
                  x0.reshape(_NP), x1.reshape(_NP),
                  r0.reshape(_NP), r1.reshape(_NP), zeros_np, ones_k)
    spr = [s.reshape(_NC, _NP) for s in sp]
    indeg = spr[4][0].reshape(_R, 128)

    sa = [s[0].reshape(_R, 128) for s in spr[:4]]
    sb = [s[1].reshape(_R, 128) for s in spr[:4]]
    q, dis2 = _tc(
        _k_mid, (sh, sh),
        sa[0], sa[1], sa[2], sa[3], sb[0], sb[1], sb[2], sb[3],
        x0, x1, r0, r1, indeg,
        g1_W, g1_b.reshape(1, 8), g2_W, smem_args=3)

    mxp = _run_max(row, col, q.reshape(_NP)).reshape(_NW, _NP)

    def g10k(v):
        return v.reshape(_NP)[:_N].reshape(_B, _N // _B)

    mall = mxp[:, :_N].reshape(_NW, _B, _N // _B)
    sm = _tc(_k_fin, jax.ShapeDtypeStruct((_B, _N // _B), f32),
             mall, g10k(q), g10k(dis2),
             g2_b.reshape(1, 1), smem_args=1)
    return (sm, imf)


# speedup vs baseline: 1.0389x; 1.0389x over previous
import functools

import jax
import jax.numpy as jnp
from jax import lax
from jax.experimental import pallas as pl
from jax.experimental.pallas import tpu as pltpu
from jax.experimental.pallas import tpu_sc as plsc

_N = 100000
_E = 3200000
_B = 10
_NC = 2
_NS = 16
_NW = _NC * _NS
_NP = 100096
_R = _NP // 128
_EPT = _E // _NW
_K = 4000
_NWIN = _EPT // _K
_SL = _NP // _NS

def _mesh():
    return plsc.VectorSubcoreMesh(
        core_axis_name="c", subcore_axis_name="s",
        num_cores=_NC, num_subcores=_NS)


def _lk(y):
    return jnp.where(y > 0, y, 0.01 * y)


def _k_mm_leaky(a_ref, b_ref, o_ref):
    o_ref[...] = _lk(jnp.dot(a_ref[...], b_ref[...],
                             preferred_element_type=jnp.float32))


def _k_mm_bias_leaky(a_ref, b_ref, c_ref, o_ref):
    o_ref[...] = _lk(jnp.dot(a_ref[...], b_ref[...],
                             preferred_element_type=jnp.float32) + c_ref[...])


def _k_maxlead(t_ref, o_ref):
    o_ref[...] = jnp.max(t_ref[...], axis=0)


def _k_head(t_ref, w_ref, b_ref, o_ref):
    m = jnp.max(t_ref[...], axis=0)
    o_ref[...] = jax.nn.sigmoid(
        jnp.dot(m, w_ref[...], preferred_element_type=jnp.float32) + b_ref[...])


def _k_norm1(d0, d1, x0, x1, r0, r1, indeg_o, dis1_o, p0_o, p1_o, p2_o, p3_o):
    indeg = d0[...] + d1[...]
    dis1 = lax.rsqrt(indeg + 1.0)
    indeg_o[...] = indeg
    dis1_o[...] = dis1
    p0_o[...] = dis1 * x0[...]
    p1_o[...] = dis1 * x1[...]
    p2_o[...] = dis1 * r0[...]
    p3_o[...] = dis1 * r1[...]


def _k_mid(sa0, sa1, sa2, sa3, sb0, sb1, sb2, sb3, x0, x1, r0, r1,
           indeg_r, dis1_r, w1_r, b1_r, w2_r, q_o, dis2_o):
    indeg = indeg_r[...]
    dis1 = dis1_r[...]
    cnt = indeg + 1.0
    h = (x0[...], x1[...], r0[...], r1[...])
    s = (sa0[...] + sb0[...], sa1[...] + sb1[...],
         sa2[...] + sb2[...], sa3[...] + sb3[...])
    agg = [dis1 * s[k] + dis1 * dis1 * h[k] for k in range(4)]
    h1w = jnp.zeros_like(indeg)
    for j in range(8):
        oj = (agg[0] * w1_r[0, j] + agg[1] * w1_r[1, j]
              + agg[2] * w1_r[2, j] + agg[3] * w1_r[3, j]) / cnt + b1_r[0, j]
        h1w = h1w + _lk(oj) * w2_r[j, 0]
    dis2 = lax.rsqrt(indeg + 2.0)
    q_o[...] = dis2 * h1w
    dis2_o[...] = dis2


def _k_fin(mall, qq, dd, b2_r, o_ref):
    z = dd[...] * jnp.maximum(jnp.max(mall[...], axis=0),
                              2.0 * qq[...]) + b2_r[0, 0]
    z = z - jnp.max(z, axis=1, keepdims=True)
    e = jnp.exp(z)
    o_ref[...] = e / jnp.sum(e, axis=1, keepdims=True)


def _tc(fn, out_shapes, *args, smem_args=0):
    n = len(args)
    in_specs = [pl.BlockSpec(memory_space=pltpu.VMEM)] * (n - smem_args) + \
               [pl.BlockSpec(memory_space=pltpu.SMEM)] * smem_args
    return pl.pallas_call(fn, out_shape=out_shapes, in_specs=in_specs)(*args)


def _wid_base():
    cid = lax.axis_index("c")
    sid = lax.axis_index("s")
    wid = sid * _NC + cid
    return cid, sid, wid * _EPT


def _deg_body(col_h, z_h, ones_h, out_h, idx_v, ones_v, obuf, acc, sem):
    cid, sid, base = _wid_base()
    o = sid * _SL
    pltpu.sync_copy(z_h.at[pl.ds(o, _SL)], obuf)
    pltpu.sync_copy(obuf, acc.at[pl.ds(o, _SL)])
    pltpu.sync_copy(ones_h, ones_v)
    plsc.subcore_barrier()

    def wb(w, carry):
        pltpu.sync_copy(col_h.at[pl.ds(base + w * _K, _K)], idx_v)
        pltpu.sync_copy(ones_v, acc.at[idx_v], add=True)
        return carry

    lax.fori_loop(0, _NWIN, wb, 0)
    plsc.subcore_barrier()
    pltpu.sync_copy(acc.at[pl.ds(o, _SL)], obuf)
    pltpu.sync_copy(obuf, out_h.at[pl.ds(cid * _NP + o, _SL)])


def _run_deg(col, zeros_np, ones_k):
    f = functools.partial(
        pl.kernel, _deg_body,
        out_type=jax.ShapeDtypeStruct((_NC * _NP,), jnp.float32),
        mesh=_mesh(),
        scratch_types=[
            pltpu.VMEM((_K,), jnp.int32),
            pltpu.VMEM((_K,), jnp.float32),
            pltpu.VMEM((_SL,), jnp.float32),
            pltpu.VMEM_SHARED((_NP,), jnp.float32),
            pltpu.SemaphoreType.DMA,
        ],
    )()
    return f(col, zeros_np, ones_k)


def _sum_body(row_h, col_h, p0_h, p1_h, p2_h, p3_h, z_h,
              o0_h, o1_h, o2_h, o3_h,
              idxr, idxc, vv0, vv1, vv2, vv3, obuf, sp0, sp1, sp2, sp3,
              ac0, ac1, ac2, ac3, sem):
    cid, sid, base = _wid_base()
    o = sid * _SL
    sl = pl.ds(o, _SL)
    for p_h, sp in ((p0_h, sp0), (p1_h, sp1), (p2_h, sp2), (p3_h, sp3)):
        pltpu.sync_copy(p_h.at[sl], obuf)
        pltpu.sync_copy(obuf, sp.at[sl])
    pltpu.sync_copy(z_h.at[sl], obuf)
    for ac in (ac0, ac1, ac2, ac3):
        pltpu.sync_copy(obuf, ac.at[sl])
    plsc.subcore_barrier()

    def wb(w, carry):
        off = base + w * _K
        pltpu.sync_copy(row_h.at[pl.ds(off, _K)], idxr)
        pltpu.sync_copy(col_h.at[pl.ds(off, _K)], idxc)
        vvs = (vv0, vv1, vv2, vv3)
        g = [pltpu.async_copy(sp.at[idxr], v, sem)
             for sp, v in zip((sp0, sp1, sp2, sp3), vvs)]
        for d in g:
            d.wait()
        s = [pltpu.async_copy(v, ac.at[idxc], sem, add=True)
             for ac, v in zip((ac0, ac1, ac2, ac3), vvs)]
        for d in s:
            d.wait()
        return carry

    lax.fori_loop(0, _NWIN, wb, 0)
    plsc.subcore_barrier()
    out_sl = pl.ds(cid * _NP + o, _SL)
    for ac, o_h in ((ac0, o0_h), (ac1, o1_h), (ac2, o2_h), (ac3, o3_h)):
        pltpu.sync_copy(ac.at[sl], obuf)
        pltpu.sync_copy(obuf, o_h.at[out_sl])


def _run_sum(row, col, p0, p1, p2, p3, zeros_np):
    st = jax.ShapeDtypeStruct((_NC * _NP,), jnp.float32)
    f = functools.partial(
        pl.kernel, _sum_body,
        out_type=(st, st, st, st),
        mesh=_mesh(),
        scratch_types=[
            pltpu.VMEM((_K,), jnp.int32),
            pltpu.VMEM((_K,), jnp.int32),
            pltpu.VMEM((_K,), jnp.float32),
            pltpu.VMEM((_K,), jnp.float32),
            pltpu.VMEM((_K,), jnp.float32),
            pltpu.VMEM((_K,), jnp.float32),
            pltpu.VMEM((_SL,), jnp.float32),
            pltpu.VMEM_SHARED((_NP,), jnp.float32),
            pltpu.VMEM_SHARED((_NP,), jnp.float32),
            pltpu.VMEM_SHARED((_NP,), jnp.float32),
            pltpu.VMEM_SHARED((_NP,), jnp.float32),
            pltpu.VMEM_SHARED((_NP,), jnp.float32),
            pltpu.VMEM_SHARED((_NP,), jnp.float32),
            pltpu.VMEM_SHARED((_NP,), jnp.float32),
            pltpu.VMEM_SHARED((_NP,), jnp.float32),
            pltpu.SemaphoreType.DMA,
        ],
    )()
    return f(row, col, p0, p1, p2, p3, zeros_np)


_SL2 = _SL // 2
_TMP = 8192


def _max_body(row_h, col_h, q_h, out_h, idxr, idxc, qv, kbuf, vbuf, tmp,
              hist, sq, abuf, sem):
    cid, sid, base = _wid_base()
    neg = jnp.float32(-3.4e38)
    o = sid * _SL
    for half in range(2):
        o2 = pl.ds(o + half * _SL2, _SL2)
        pltpu.sync_copy(q_h.at[o2], abuf)
        pltpu.sync_copy(abuf, sq.at[o2])
    plsc.subcore_barrier()

    def ib(j, carry):
        hist[pl.ds(j * 16, 16)] = jnp.full((16,), neg, jnp.float32)
        return carry

    lax.fori_loop(0, _NP // 16, ib, 0)

    def wb(w, carry):
        off = base + w * _K
        pltpu.sync_copy(row_h.at[pl.ds(off, _K)], idxr)
        pltpu.sync_copy(col_h.at[pl.ds(off, _K)], idxc)
        pltpu.async_copy(sq.at[idxr], qv, sem).wait()

        lane = lax.iota(jnp.int32, 16)

        def vb(j, c2):
            ck = idxc[pl.ds(j * 16, 16)]
            vv = qv[pl.ds(j * 16, 16)]
            hk = lax.bitwise_and(ck, _TMP - 1)
            lane_f = lane.astype(jnp.float32)
            plsc.store_scatter(tmp, [hk], lane_f)
            w = plsc.load_gather(tmp, [hk])
            dup = jnp.any(w != lane_f)

            @pl.when(jnp.logical_not(dup))
            def _():
                cur = plsc.load_gather(hist, [ck])
                plsc.store_scatter(hist, [ck], jnp.maximum(cur, vv))

            @pl.when(dup)
            def _():
                ck_f = ck.astype(jnp.float32)
                kbuf[...] = ck_f
                vbuf[...] = vv
                acc = vv
                for r in range(1, 16):
                    ir = lax.bitwise_and(lane + r, 15)
                    kr = plsc.load_gather(kbuf, [ir])
                    vr = plsc.load_gather(vbuf, [ir])
                    acc = jnp.where(kr == ck_f, jnp.maximum(acc, vr), acc)
                cur = plsc.load_gather(hist, [ck])
                plsc.store_scatter(hist, [ck], jnp.maximum(cur, acc))

            return c2

        lax.fori_loop(0, _K // 16, vb, 0)
        return carry

    lax.fori_loop(0, _NWIN, wb, 0)
    wid = sid * _NC + cid
    pltpu.sync_copy(hist, out_h.at[pl.ds(wid * _NP, _NP)])


def _run_max(row, col, qtab):
    f = functools.partial(
        pl.kernel, _max_body,
        out_type=jax.ShapeDtypeStruct((_NW * _NP,), jnp.float32),
        mesh=_mesh(),
        compiler_params=pltpu.CompilerParams(needs_layout_passes=False),
        scratch_types=[
            pltpu.VMEM((_K,), jnp.int32),
            pltpu.VMEM((_K,), jnp.int32),
            pltpu.VMEM((_K,), jnp.float32),
            pltpu.VMEM((16,), jnp.float32),
            pltpu.VMEM((16,), jnp.float32),
            pltpu.VMEM((_TMP,), jnp.float32),
            pltpu.VMEM((_NP,), jnp.float32),
            pltpu.VMEM_SHARED((_NP,), jnp.float32),
            pltpu.VMEM((_SL2,), jnp.float32),
            pltpu.SemaphoreType.DMA,
        ],
    )()
    return f(row, col, qtab)


def kernel(im, x, edge_index, num_nodes_per_graph, conv1_w, conv2_w, conv2_b,
           lin_w, lin_b, g1_W, g1_b, g2_W, g2_b):
    f32 = jnp.float32

    im_p = jnp.pad(im, ((0, 0), (0, 0), (2, 2), (2, 2)))
    t1 = [im_p[:, :, ky:ky + 57:4, kx:kx + 57:4]
          for ky in range(11) for kx in range(11)]
    pat1 = jnp.stack(t1, 0).transpose(1, 3, 4, 2, 0).reshape(_B * 225, 363)
    c1 = _tc(_k_mm_leaky, jax.ShapeDtypeStruct((_B * 225, 10), f32),
             pat1, conv1_w.reshape(10, 363).T)
    c1r = c1.reshape(_B, 15, 15, 10)
    t2 = [c1r[:, dy:dy + 13:2, dx:dx + 13:2, :]
          for dy in range(3) for dx in range(3)]
    p1 = _tc(_k_maxlead, jax.ShapeDtypeStruct((_B * 49, 10), f32),
             jnp.stack(t2, 0).reshape(9, _B * 49, 10))
    p1p = jnp.pad(p1.reshape(_B, 7, 7, 10).transpose(0, 3, 1, 2),
                  ((0, 0), (0, 0), (2, 2), (2, 2)))
    t3 = [p1p[:, :, ky:ky + 7:2, kx:kx + 7:2]
          for ky in range(5) for kx in range(5)]
    pat2 = jnp.stack(t3, 0).transpose(1, 3, 4, 2, 0).reshape(_B * 16, 250)
    c2 = _tc(_k_mm_bias_leaky, jax.ShapeDtypeStruct((_B * 16, 16), f32),
             pat2, conv2_w.reshape(16, 250).T, conv2_b.reshape(1, 16))
    c2r = c2.reshape(_B, 4, 4, 16)
    t4 = [c2r[:, dy:dy + 3:2, dx:dx + 3:2, :].transpose(0, 3, 1, 2).reshape(_B, 64)
          for dy in range(2) for dx in range(2)]
    imf = _tc(_k_head, jax.ShapeDtypeStruct((_B, 2), f32),
              jnp.stack(t4, 0), lin_w.T, lin_b.reshape(1, 2))

    row = edge_index[0]
    col = edge_index[1]
    rep = jnp.repeat(imf, num_nodes_per_graph, axis=0, total_repeat_length=_N)
    pad = _NP - _N

    def nshape(v):
        return jnp.pad(v, (0, pad)).reshape(_R, 128)

    x0, x1 = nshape(x[:, 0]), nshape(x[:, 1])
    r0, r1 = nshape(rep[:, 0]), nshape(rep[:, 1])

    zeros_np = jnp.zeros((_NP,), f32)
    ones_k = jnp.ones((_K,), f32)

    degp = _run_deg(col, zeros_np, ones_k).reshape(_NC, _NP)

    sh = jax.ShapeDtypeStruct((_R, 128), f32)
    indeg, dis1, p0, p1_, p2, p3 = _tc(
        _k_norm1, (sh,) * 6,
        degp[0].reshape(_R, 128), degp[1].reshape(_R, 128), x0, x1, r0, r1)

    sp = _run_sum(row, col, p0.reshape(_NP), p1_.reshape(_NP),
                  p2.reshape(_NP), p3.reshape(_NP), zeros_np)
    spr = [s.reshape(_NC, _NP) for s in sp]

    sa = [s[0].reshape(_R, 128) for s in spr]
    sb = [s[1].reshape(_R, 128) for s in spr]
    q, dis2 = _tc(
        _k_mid, (sh, sh),
        sa[0], sa[1], sa[2], sa[3], sb[0], sb[1], sb[2], sb[3],
        x0, x1, r0, r1, indeg, dis1,
        g1_W, g1_b.reshape(1, 8), g2_W, smem_args=3)

    mxp = _run_max(row, col, q.reshape(_NP)).reshape(_NW, _NP)

    def g10k(v):
        return v.reshape(_NP)[:_N].reshape(_B, _N // _B)

    mall = mxp[:, :_N].reshape(_NW, _B, _N // _B)
    sm = _tc(_k_fin, jax.ShapeDtypeStruct((_B, _N // _B), f32),
             mall, g10k(q), g10k(dis2),
             g2_b.reshape(1, 1), smem_args=1)
    return (sm, imf)
